# baseline (speedup 1.0000x reference)
import jax
import jax.numpy as jnp
from jax import lax
from jax.experimental import pallas as pl
from jax.experimental.pallas import tpu as pltpu
from jax.experimental.pallas import tpu_sc as plsc

NC = 2
NS = 16
NW = NC * NS
CHUNK = 100
NBUF = 4


def _body(x_hbm, table_hbm, out_hbm, idx_v, bufs, gsems, wsems):
    wid = lax.axis_index("s") * NC + lax.axis_index("c")
    nchunks = x_hbm.shape[1]
    ngroups = nchunks // NBUF

    pltpu.sync_copy(x_hbm.at[wid], idx_v)

    def gather(c, b):
        pltpu.make_async_copy(
            table_hbm.at[idx_v.at[c]], bufs[b], gsems[b]).start()

    def wait_gather(b):
        pltpu.make_async_copy(
            table_hbm.at[idx_v.at[0]], bufs[b], gsems[b]).wait()

    def write(c, b):
        pltpu.make_async_copy(bufs[b], out_hbm.at[wid, c], wsems[b]).start()

    def wait_write(b):
        pltpu.make_async_copy(bufs[b], out_hbm.at[wid, 0], wsems[b]).wait()

    for b in range(NBUF):
        gather(b, b)

    def step(g, carry):
        c0 = NBUF * g
        for b in range(NBUF):
            wait_gather(b)
            write(c0 + b, b)
        for b in range(NBUF):
            wait_write(b)
            gather(c0 + NBUF + b, b)
        return carry

    lax.fori_loop(0, ngroups - 1, step, 0)

    c0 = (ngroups - 1) * NBUF
    for b in range(NBUF):
        wait_gather(b)
        write(c0 + b, b)
    for b in range(NBUF):
        wait_write(b)


def kernel(x, table):
    B0, B1 = x.shape
    V, D = table.shape
    total = B0 * B1
    nchunks = total // (NW * CHUNK)
    x3 = x.reshape(NW, nchunks, CHUNK)

    fn = pl.kernel(
        _body,
        out_type=jax.ShapeDtypeStruct((NW, nchunks, CHUNK, D), jnp.float32),
        mesh=plsc.VectorSubcoreMesh(core_axis_name="c", subcore_axis_name="s"),
        compiler_params=pltpu.CompilerParams(use_tc_tiling_on_sc=False),
        scratch_types=[
            pltpu.VMEM((nchunks, CHUNK), jnp.int32),
            [pltpu.VMEM((CHUNK, D), jnp.float32) for _ in range(NBUF)],
            [pltpu.SemaphoreType.DMA for _ in range(NBUF)],
            [pltpu.SemaphoreType.DMA for _ in range(NBUF)],
        ],
    )
    out4 = fn(x3, table)
    return out4.reshape(B0, B1, D)


# speedup vs baseline: 1.7406x; 1.7406x over previous
import jax
import jax.numpy as jnp
from jax import lax
from jax.experimental import pallas as pl
from jax.experimental.pallas import tpu as pltpu
from jax.experimental.pallas import tpu_sc as plsc

NC = 2
NS = 16
NW = NC * NS
CHUNK = 100
NBUF = 4


def _body(x_hbm, table_hbm, out_hbm, idx_v, table_v, table_sh, bufs,
          gsems, wsems):
    sid = lax.axis_index("s")
    wid = sid * NC + lax.axis_index("c")
    nchunks = x_hbm.shape[1]
    ngroups = nchunks // NBUF

    @pl.when(sid == 0)
    def _():
        pltpu.sync_copy(table_hbm, table_v)
        pltpu.sync_copy(table_v, table_sh)

    pltpu.sync_copy(x_hbm.at[wid], idx_v)
    plsc.subcore_barrier()

    def gather(c, b):
        pltpu.make_async_copy(
            table_sh.at[idx_v.at[c]], bufs[b], gsems[b]).start()

    def wait_gather(b):
        pltpu.make_async_copy(
            table_sh.at[idx_v.at[0]], bufs[b], gsems[b]).wait()

    def write(c, b):
        pltpu.make_async_copy(bufs[b], out_hbm.at[wid, c], wsems[b]).start()

    def wait_write(b):
        pltpu.make_async_copy(bufs[b], out_hbm.at[wid, 0], wsems[b]).wait()

    for b in range(NBUF):
        gather(b, b)

    def step(g, carry):
        c0 = NBUF * g
        for b in range(NBUF):
            wait_gather(b)
            write(c0 + b, b)
        for b in range(NBUF):
            wait_write(b)
            gather(c0 + NBUF + b, b)
        return carry

    lax.fori_loop(0, ngroups - 1, step, 0)

    c0 = (ngroups - 1) * NBUF
    for b in range(NBUF):
        wait_gather(b)
        write(c0 + b, b)
    for b in range(NBUF):
        wait_write(b)


def kernel(x, table):
    B0, B1 = x.shape
    V, D = table.shape
    total = B0 * B1
    nchunks = total // (NW * CHUNK)
    x3 = x.reshape(NW, nchunks, CHUNK)

    fn = pl.kernel(
        _body,
        out_type=jax.ShapeDtypeStruct((NW, nchunks, CHUNK, D), jnp.float32),
        mesh=plsc.VectorSubcoreMesh(core_axis_name="c", subcore_axis_name="s"),
        compiler_params=pltpu.CompilerParams(use_tc_tiling_on_sc=False),
        scratch_types=[
            pltpu.VMEM((nchunks, CHUNK), jnp.int32),
            pltpu.VMEM((V, D), jnp.float32),
            pltpu.VMEM_SHARED((V, D), jnp.float32),
            [pltpu.VMEM((CHUNK, D), jnp.float32) for _ in range(NBUF)],
            [pltpu.SemaphoreType.DMA for _ in range(NBUF)],
            [pltpu.SemaphoreType.DMA for _ in range(NBUF)],
        ],
    )
    out4 = fn(x3, table)
    return out4.reshape(B0, B1, D)
